# baseline (speedup 1.0000x reference)
import math

import jax
import jax.numpy as jnp
from jax.experimental import pallas as pl
from jax.experimental.pallas import tpu as pltpu

N = 4096
NFEAT = 256
NHID = 64
NLAYERS = 4
LAMDA = 0.5
ALPHA = 0.1

NB = 16
BLK = N // NB
CBLK = 1024


def _theta(i):
    return math.log(LAMDA / (i + 1) + 1.0)


def _layer_block(adj_bf, cur_bf, h0_rows, wi_bf, i):
    th = _theta(i)
    k2 = adj_bf.shape[1] // 2
    hi = jnp.dot(adj_bf[:, :k2], cur_bf[:k2],
                 preferred_element_type=jnp.float32) \
        + jnp.dot(adj_bf[:, k2:], cur_bf[k2:],
                  preferred_element_type=jnp.float32)
    support = (1.0 - ALPHA) * hi + ALPHA * h0_rows
    out = th * jnp.dot(support.astype(jnp.bfloat16), wi_bf,
                       preferred_element_type=jnp.float32) \
        + (1.0 - th) * support
    return jnp.maximum(out, 0.0)


def _gcnii_body(x_ref, adj_ref, w0_ref, b_ref, cw_ref, out_ref,
                abf_ref, h0_ref, ca_ref, cb_ref):
    j = pl.program_id(0)

    @pl.when(j == 0)
    def _():
        xb = x_ref[...].astype(jnp.bfloat16)
        w0 = w0_ref[...].astype(jnp.bfloat16)
        h0_ref[...] = jnp.maximum(
            jnp.dot(xb, w0, preferred_element_type=jnp.float32) + b_ref[...],
            0.0)

    rows = pl.ds(j * BLK, BLK)
    blk_bf = adj_ref[...].astype(jnp.bfloat16)
    abf_ref[rows, :] = blk_bf
    h0_bf = h0_ref[...].astype(jnp.bfloat16)
    ca_ref[rows, :] = _layer_block(blk_bf, h0_bf, h0_ref[rows, :],
                                   cw_ref[0].astype(jnp.bfloat16), 0)

    @pl.when(j == NB - 1)
    def _():
        for i in range(1, NLAYERS):
            src = ca_ref if i % 2 == 1 else cb_ref
            dst = None if i == NLAYERS - 1 else (cb_ref if i % 2 == 1 else ca_ref)
            cur_bf = src[...].astype(jnp.bfloat16)
            wi_bf = cw_ref[i].astype(jnp.bfloat16)
            for jj in range(N // CBLK):
                r = pl.ds(jj * CBLK, CBLK)
                res = _layer_block(abf_ref[r, :], cur_bf, h0_ref[r, :],
                                   wi_bf, i)
                if dst is None:
                    out_ref[r, NFEAT:] = res
                else:
                    dst[r, :] = res
        out_ref[:, :NFEAT] = x_ref[...]


def kernel(x, adj, fc0_w, fc0_b, conv_w):
    return pl.pallas_call(
        _gcnii_body,
        grid=(NB,),
        in_specs=[
            pl.BlockSpec((N, NFEAT), lambda j: (0, 0)),
            pl.BlockSpec((BLK, N), lambda j: (j, 0)),
            pl.BlockSpec((NFEAT, NHID), lambda j: (0, 0)),
            pl.BlockSpec((1, NHID), lambda j: (0, 0)),
            pl.BlockSpec((NLAYERS, NHID, NHID), lambda j: (0, 0, 0)),
        ],
        out_specs=pl.BlockSpec((N, NFEAT + NHID), lambda j: (0, 0)),
        out_shape=jax.ShapeDtypeStruct((N, NFEAT + NHID), jnp.float32),
        scratch_shapes=[
            pltpu.VMEM((N, N), jnp.bfloat16),
            pltpu.VMEM((N, NHID), jnp.float32),
            pltpu.VMEM((N, NHID), jnp.float32),
            pltpu.VMEM((N, NHID), jnp.float32),
        ],
    )(x, adj, fc0_w, fc0_b.reshape(1, NHID), conv_w)


# speedup vs baseline: 1.1903x; 1.1903x over previous
import math

import jax
import jax.numpy as jnp
from jax.experimental import pallas as pl
from jax.experimental.pallas import tpu as pltpu

N = 4096
NFEAT = 256
NHID = 64
NLAYERS = 4
LAMDA = 0.5
ALPHA = 0.1

NB = 32
BLK = N // NB
CBLK = 1024


def _gcnii_body(x_ref, adj_ref, w0_ref, b_ref, bw_ref, out_ref,
                abf_ref, h0_ref, cur_ref, t_ref, g_ref):
    j = pl.program_id(0)

    @pl.when(j == 0)
    def _():
        xb = x_ref[...].astype(jnp.bfloat16)
        w0 = w0_ref[...].astype(jnp.bfloat16)
        h0 = jnp.maximum(
            jnp.dot(xb, w0, preferred_element_type=jnp.float32) + b_ref[...],
            0.0)
        h0_ref[...] = h0
        hb = h0.astype(jnp.bfloat16)
        s = jnp.dot(hb, bw_ref[0].astype(jnp.bfloat16),
                    preferred_element_type=jnp.float32)
        t_ref[...] = s
        g_ref[...] = s * (1.0 / 9.0)

    rows = pl.ds(j * BLK, BLK)
    blk_bf = adj_ref[...].astype(jnp.bfloat16)
    abf_ref[rows, :] = blk_bf
    t_bf = t_ref[...].astype(jnp.bfloat16)
    cur_ref[rows, :] = jnp.maximum(
        jnp.dot(blk_bf, t_bf, preferred_element_type=jnp.float32)
        + g_ref[rows, :], 0.0)

    @pl.when(j == NB - 1)
    def _():
        for i in range(1, NLAYERS):
            cb = cur_ref[...].astype(jnp.bfloat16)
            bw = bw_ref[i].astype(jnp.bfloat16)
            s = jnp.dot(cb, bw, preferred_element_type=jnp.float32)
            t_bf2 = s.astype(jnp.bfloat16)
            g = jnp.dot(h0_ref[...].astype(jnp.bfloat16), bw,
                        preferred_element_type=jnp.float32) * (1.0 / 9.0)
            g_ref[...] = g
            for jj in range(N // CBLK):
                r = pl.ds(jj * CBLK, CBLK)
                res = jnp.maximum(
                    jnp.dot(abf_ref[r, :], t_bf2,
                            preferred_element_type=jnp.float32)
                    + g_ref[r, :], 0.0)
                if i == NLAYERS - 1:
                    out_ref[r, NFEAT:] = res
                else:
                    cur_ref[r, :] = res
        out_ref[:, :NFEAT] = x_ref[...]


def kernel(x, adj, fc0_w, fc0_b, conv_w):
    thetas = jnp.array([math.log(LAMDA / (i + 1) + 1.0)
                        for i in range(NLAYERS)], dtype=jnp.float32)
    eye = jnp.eye(NHID, dtype=jnp.float32)
    bw = (1.0 - ALPHA) * (thetas[:, None, None] * conv_w
                          + (1.0 - thetas)[:, None, None] * eye[None])
    return pl.pallas_call(
        _gcnii_body,
        grid=(NB,),
        in_specs=[
            pl.BlockSpec((N, NFEAT), lambda j: (0, 0)),
            pl.BlockSpec((BLK, N), lambda j: (j, 0)),
            pl.BlockSpec((NFEAT, NHID), lambda j: (0, 0)),
            pl.BlockSpec((1, NHID), lambda j: (0, 0)),
            pl.BlockSpec((NLAYERS, NHID, NHID), lambda j: (0, 0, 0)),
        ],
        out_specs=pl.BlockSpec((N, NFEAT + NHID), lambda j: (0, 0)),
        out_shape=jax.ShapeDtypeStruct((N, NFEAT + NHID), jnp.float32),
        scratch_shapes=[
            pltpu.VMEM((N, N), jnp.bfloat16),
            pltpu.VMEM((N, NHID), jnp.float32),
            pltpu.VMEM((N, NHID), jnp.float32),
            pltpu.VMEM((N, NHID), jnp.float32),
            pltpu.VMEM((N, NHID), jnp.float32),
        ],
    )(x, adj, fc0_w, fc0_b.reshape(1, NHID), bw)
